# baseline (speedup 1.0000x reference)
import functools

import jax
import jax.numpy as jnp
from jax import lax
from jax.experimental import pallas as pl
from jax.experimental.pallas import tpu as pltpu
from jax.experimental.pallas import tpu_sc as plsc

N = 50000
E = 800000
G = 1024
D = 64
NCONV = 3
RBF = 5
GAP = 1.25
NB = 2000
EB2 = 2000
CH = 128
NCH = E // CH
NSUB = 16
RPT = 3128
NPAD = NSUB * RPT
CW = 16
CG = 4


def _sp(x, beta, thr):
    z = x * beta
    return jnp.where(z > thr, x,
                     (1.0 / beta) * jnp.log1p(jnp.exp(jnp.minimum(z, thr))))


def _emb_body(at_ref, emb_ref, w_ref, h_ref, nn_ref):
    at = at_ref[:]
    oh = (at == lax.broadcasted_iota(jnp.int32, (NB, 100), 1))
    h = jnp.dot(oh.astype(jnp.float32), emb_ref[:],
                preferred_element_type=jnp.float32)
    h_ref[:] = h
    nn = jnp.dot(h, w_ref[:], preferred_element_type=jnp.float32)
    for g in range(CG):
        nn_ref[g] = nn[:, g * CW:(g + 1) * CW]


def _eh_body(d_ref, w1b_ref, b1t_ref, w2b_ref, b2t_ref, out_ref):
    d16 = d_ref[:]
    ctile = (lax.broadcasted_iota(jnp.int32, (1, 16), 1) & 7
             ).astype(jnp.float32) * GAP
    radial = d16 - ctile
    rbf = jnp.exp((-1.0 / GAP) * radial * radial)
    p = _sp(jnp.dot(rbf, w1b_ref[:], preferred_element_type=jnp.float32)
            + b1t_ref[:], 0.5, 14.0)
    eh = jnp.dot(p, w2b_ref[:], preferred_element_type=jnp.float32) + b2t_ref[:]
    out_ref[:] = eh


def _upd_body(a_ref, h_ref, wn2_ref, bn2_ref, wn3_ref, bn3_ref,
              hout_ref, wnext_ref=None, nnout_ref=None):
    a = jnp.concatenate([a_ref[g] for g in range(CG)], axis=1)
    cf1 = _sp(jnp.dot(a, wn2_ref[:], preferred_element_type=jnp.float32)
              + bn2_ref[:], 0.5, 14.0)
    hn = h_ref[:] + jnp.dot(cf1, wn3_ref[:],
                            preferred_element_type=jnp.float32) + bn3_ref[:]
    hout_ref[:] = hn
    if wnext_ref is not None:
        nn = jnp.dot(hn, wnext_ref[:], preferred_element_type=jnp.float32)
        for g in range(CG):
            nnout_ref[g] = nn[:, g * CW:(g + 1) * CW]


def _upd_next_body(a_ref, h_ref, wn2_ref, bn2_ref, wn3_ref, bn3_ref,
                   wnext_ref, hout_ref, nnout_ref):
    _upd_body(a_ref, h_ref, wn2_ref, bn2_ref, wn3_ref, bn3_ref,
              hout_ref, wnext_ref, nnout_ref)


def _upd_last_body(a_ref, h_ref, wn2_ref, bn2_ref, wn3_ref, bn3_ref, hout_ref):
    _upd_body(a_ref, h_ref, wn2_ref, bn2_ref, wn3_ref, bn3_ref, hout_ref)


def _out_body(h_ref, gid_ref, wu1_ref, bu1_ref, wu2_ref, bu2_ref, out_ref):
    i = pl.program_id(0)
    hh = _sp(jnp.dot(h_ref[:], wu1_ref[:], preferred_element_type=jnp.float32)
             + bu1_ref[:], 1.0, 20.0) - jnp.log(2.0)
    u = jnp.dot(hh, wu2_ref[:], preferred_element_type=jnp.float32) + bu2_ref[:]
    gid = gid_ref[:]
    oh = (gid == lax.broadcasted_iota(jnp.int32, (NB, G), 1))
    contrib = lax.dot_general(oh.astype(jnp.float32), u,
                              dimension_numbers=(((0,), (0,)), ((), ())),
                              preferred_element_type=jnp.float32)

    @pl.when(i == 0)
    def _():
        out_ref[:] = jnp.zeros_like(out_ref)

    out_ref[:] = out_ref[:] + contrib


NBUF = 8
ZR = 184


def _edge_body(nn, ehh, src2, dst2, out, sidx, didx, sdidx, rows, ehb, zbuf,
               agg, sem_ld, sem_g, sem_s):
    c = lax.axis_index("c")
    s = lax.axis_index("s")
    z16 = jnp.zeros((16,), jnp.float32)

    def zb(r, carry):
        zbuf[r, pl.ds(0, 16)] = z16
        return carry
    lax.fori_loop(0, ZR, zb, 0)

    base = NCH // NSUB
    extra = NCH - base * NSUB
    start = s * base + jnp.minimum(s, extra)
    cnt = base + (s < extra).astype(jnp.int32)
    n_sc = cnt // NBUF
    rem = cnt - n_sc * NBUF
    npairs = n_sc // 2
    n_pipe = npairs * 2

    HC = NBUF * CH // 2

    def issue_loads(t, p):
        ch0 = start + t * NBUF
        pltpu.async_copy(src2.at[pl.ds(ch0, NBUF)], sidx.at[p], sem_ld)
        pltpu.async_copy(dst2.at[pl.ds(ch0, NBUF)], didx.at[p], sem_ld)
        r0 = ch0 * (CH // 2)
        pltpu.async_copy(ehh.at[pl.ds(r0, HC), pl.ds(g * CW, CW)],
                         ehb.at[p, 0], sem_ld)
        pltpu.async_copy(ehh.at[pl.ds(r0, HC), pl.ds(D + g * CW, CW)],
                         ehb.at[p, 1], sem_ld)

    def wait_loads(p):
        pltpu.make_async_copy(src2.at[pl.ds(0, NBUF)], sidx.at[p],
                              sem_ld).wait()
        pltpu.make_async_copy(dst2.at[pl.ds(0, NBUF)], didx.at[p],
                              sem_ld).wait()
        pltpu.make_async_copy(ehh.at[pl.ds(0, HC), pl.ds(0, CW)],
                              ehb.at[p, 0], sem_ld).wait()
        pltpu.make_async_copy(ehh.at[pl.ds(0, HC), pl.ds(0, CW)],
                              ehb.at[p, 1], sem_ld).wait()

    def adjust(p):
        for b in range(NBUF):
            for j in range(CH // 16):
                sl = pl.ds(j * 16, 16)
                sidx[p, b, sl] = sidx[p, b, sl] + goff

    def issue_gathers(p):
        for b in range(NBUF):
            pltpu.async_copy(nn.at[sidx.at[p, b]], rows.at[p, b], sem_g)

    def wait_gathers(p):
        for b in range(NBUF):
            pltpu.make_async_copy(nn.at[sidx.at[p, b]], rows.at[p, b],
                                  sem_g).wait()

    def do_mul(p):
        sl = pl.ds(0, 16)

        def mul(rr, cc):
            for b in range(NBUF):
                q = b * (CH // 2) + rr
                rows[p, b, 2 * rr, sl] = (rows[p, b, 2 * rr, sl]
                                          * ehb[p, 0, q, sl])
                rows[p, b, 2 * rr + 1, sl] = (rows[p, b, 2 * rr + 1, sl]
                                              * ehb[p, 1, q, sl])
            return cc
        lax.fori_loop(0, CH // 2, mul, 0)

    def issue_scatters(p):
        for b in range(NBUF):
            for j in range(CH // 16):
                sl = pl.ds(j * 16, 16)
                sdidx[p, b, sl] = didx[p, b, sl]
        for b in range(NBUF):
            pltpu.async_copy(rows.at[p, b], agg.at[sdidx.at[p, b]], sem_s,
                             add=True)

    def drain_scatters(p):
        for b in range(NBUF):
            pltpu.make_async_copy(rows.at[p, b], agg.at[sdidx.at[p, b]],
                                  sem_s).wait()

    def step(t, p):
        wait_gathers(p)
        do_mul(p)
        issue_scatters(p)

        @pl.when(t + 1 < n_pipe)
        def _():
            wait_loads(1 - p)
            adjust(1 - p)

        @pl.when(t >= 1)
        def _():
            drain_scatters(1 - p)

        @pl.when(t + 1 < n_pipe)
        def _():
            issue_gathers(1 - p)

        @pl.when(t + 2 < n_pipe)
        def _():
            issue_loads(t + 2, p)

    def do_chunks(ch0, nb):
        r0 = ch0 * (CH // 2)
        pltpu.sync_copy(src2.at[pl.ds(ch0, nb)], sidx.at[0, pl.ds(0, nb)])
        pltpu.sync_copy(dst2.at[pl.ds(ch0, nb)], didx.at[0, pl.ds(0, nb)])
        pltpu.sync_copy(ehh.at[pl.ds(r0, nb * CH // 2), pl.ds(g * CW, CW)],
                        ehb.at[0, 0, pl.ds(0, nb * CH // 2)])
        pltpu.sync_copy(ehh.at[pl.ds(r0, nb * CH // 2),
                               pl.ds(D + g * CW, CW)],
                        ehb.at[0, 1, pl.ds(0, nb * CH // 2)])
        for b in range(nb):
            for j in range(CH // 16):
                sl = pl.ds(j * 16, 16)
                sidx[0, b, sl] = sidx[0, b, sl] + goff
        gds = [pltpu.async_copy(nn.at[sidx.at[0, b]], rows.at[0, b], sem_g)
               for b in range(nb)]
        for d in gds:
            d.wait()
        sl16 = pl.ds(0, 16)
        for b in range(nb):
            def mul(rr, cc):
                q = b * (CH // 2) + rr
                rows[0, b, 2 * rr, sl16] = (rows[0, b, 2 * rr, sl16]
                                            * ehb[0, 0, q, sl16])
                rows[0, b, 2 * rr + 1, sl16] = (rows[0, b, 2 * rr + 1, sl16]
                                                * ehb[0, 1, q, sl16])
                return cc
            lax.fori_loop(0, CH // 2, mul, 0)
        sds = [pltpu.async_copy(rows.at[0, b], agg.at[didx.at[0, b]], sem_s,
                                add=True)
               for b in range(nb)]
        for d in sds:
            d.wait()

    for cp in range(CG // 2):
        g = c * (CG // 2) + cp
        goff = g * N
        ooff = g * NPAD

        def zs(k, carry):
            pltpu.sync_copy(zbuf, agg.at[pl.ds(s * RPT + k * ZR, ZR)])
            return carry
        lax.fori_loop(0, RPT // ZR, zs, 0)
        plsc.subcore_barrier()

        issue_loads(0, 0)
        wait_loads(0)
        adjust(0)
        issue_gathers(0)

        @pl.when(n_pipe > 1)
        def _():
            issue_loads(1, 1)

        def per_pair(q, carry):
            step(2 * q, 0)
            step(2 * q + 1, 1)
            return carry
        lax.fori_loop(0, npairs, per_pair, 0)
        drain_scatters(1)

        def per_odd(t, carry):
            do_chunks(start + n_pipe * NBUF, NBUF)
            return carry
        lax.fori_loop(0, n_sc - n_pipe, per_odd, 0)

        def per_tail(t, carry):
            do_chunks(start + n_sc * NBUF + t, 1)
            return carry
        lax.fori_loop(0, rem, per_tail, 0)

        plsc.subcore_barrier()
        pltpu.sync_copy(agg.at[pl.ds(s * RPT, RPT)],
                        out.at[pl.ds(ooff + s * RPT, RPT)])


_edge_pass = functools.partial(
    pl.kernel,
    out_type=jax.ShapeDtypeStruct((CG * NPAD, CW), jnp.float32),
    mesh=plsc.VectorSubcoreMesh(core_axis_name="c", subcore_axis_name="s",
                                num_cores=2, num_subcores=NSUB),
    compiler_params=pltpu.CompilerParams(use_tc_tiling_on_sc=False),
    scratch_types=[
        pltpu.VMEM((2, NBUF, CH), jnp.int32),
        pltpu.VMEM((2, NBUF, CH), jnp.int32),
        pltpu.VMEM((2, NBUF, CH), jnp.int32),
        pltpu.VMEM((2, NBUF, CH, CW), jnp.float32),
        pltpu.VMEM((2, 2, NBUF * CH // 2, CW), jnp.float32),
        pltpu.VMEM((ZR, CW), jnp.float32),
        pltpu.VMEM_SHARED((NPAD, CW), jnp.float32),
        pltpu.SemaphoreType.DMA,
        pltpu.SemaphoreType.DMA,
        pltpu.SemaphoreType.DMA,
    ],
)(_edge_body)


def _full(shape):
    return pl.BlockSpec(shape, lambda i: tuple(0 for _ in shape))


def kernel(atom_types, edge_distances, edge_index, node_graph_ids, emb,
           Wn1, Wc1, bc1, Wc2, bc2, Wn2, bn2, Wn3, bn3, Wu1, bu1, Wu2, bu2):
    f32 = jnp.float32
    src = edge_index[0].astype(jnp.int32)
    dst = edge_index[1].astype(jnp.int32)
    at = atom_types.astype(jnp.int32).reshape(N, 1)
    gids = node_graph_ids.astype(jnp.int32).reshape(N, 1)

    h, nn = pl.pallas_call(
        _emb_body,
        grid=(N // NB,),
        in_specs=[pl.BlockSpec((NB, 1), lambda i: (i, 0)),
                  _full((100, D)),
                  _full((D, D))],
        out_specs=[pl.BlockSpec((NB, D), lambda i: (i, 0)),
                   pl.BlockSpec((CG, NB, CW), lambda i: (0, i, 0))],
        out_shape=[jax.ShapeDtypeStruct((N, D), f32),
                   jax.ShapeDtypeStruct((CG, N, CW), f32)],
    )(at, emb, Wn1[0])

    d2 = jnp.repeat(edge_distances.reshape(E // 2, 2), 8, axis=1)
    z16 = jnp.zeros((16, 2 * D), f32)
    z128 = jnp.zeros((2 * D, 2 * D), f32)
    for i in range(NCONV):
        w1b = z16.at[0:RBF, 0:D].set(Wc1[i]).at[8:8 + RBF, D:2 * D].set(Wc1[i])
        w2b = z128.at[0:D, 0:D].set(Wc2[i]).at[D:2 * D, D:2 * D].set(Wc2[i])
        b1t = jnp.concatenate([bc1[i], bc1[i]]).reshape(1, 2 * D)
        b2t = jnp.concatenate([bc2[i], bc2[i]]).reshape(1, 2 * D)
        eh2 = pl.pallas_call(
            _eh_body,
            grid=(E // 2 // EB2,),
            in_specs=[pl.BlockSpec((EB2, 16), lambda k: (k, 0)),
                      _full((16, 2 * D)),
                      _full((1, 2 * D)),
                      _full((2 * D, 2 * D)),
                      _full((1, 2 * D))],
            out_specs=pl.BlockSpec((EB2, 2 * D), lambda k: (k, 0)),
            out_shape=jax.ShapeDtypeStruct((E // 2, 2 * D), f32),
        )(d2, w1b, b1t, w2b, b2t)

        agg_flat = _edge_pass(nn.reshape(CG * N, CW), eh2,
                              src.reshape(NCH, CH), dst.reshape(NCH, CH))
        agg = agg_flat.reshape(CG, NPAD, CW)

        if i + 1 < NCONV:
            h, nn = pl.pallas_call(
                _upd_next_body,
                grid=(N // NB,),
                in_specs=[pl.BlockSpec((CG, NB, CW), lambda k: (0, k, 0)),
                          pl.BlockSpec((NB, D), lambda k: (k, 0)),
                          _full((D, D)), _full((1, D)),
                          _full((D, D)), _full((1, D)),
                          _full((D, D))],
                out_specs=[pl.BlockSpec((NB, D), lambda k: (k, 0)),
                           pl.BlockSpec((CG, NB, CW), lambda k: (0, k, 0))],
                out_shape=[jax.ShapeDtypeStruct((N, D), f32),
                           jax.ShapeDtypeStruct((CG, N, CW), f32)],
            )(agg, h, Wn2[i], bn2[i].reshape(1, D),
              Wn3[i], bn3[i].reshape(1, D), Wn1[i + 1])
        else:
            h = pl.pallas_call(
                _upd_last_body,
                grid=(N // NB,),
                in_specs=[pl.BlockSpec((CG, NB, CW), lambda k: (0, k, 0)),
                          pl.BlockSpec((NB, D), lambda k: (k, 0)),
                          _full((D, D)), _full((1, D)),
                          _full((D, D)), _full((1, D))],
                out_specs=pl.BlockSpec((NB, D), lambda k: (k, 0)),
                out_shape=jax.ShapeDtypeStruct((N, D), f32),
            )(agg, h, Wn2[i], bn2[i].reshape(1, D),
              Wn3[i], bn3[i].reshape(1, D))

    wu2p = jnp.pad(Wu2, ((0, 0), (0, 7)))
    bu2p = jnp.pad(bu2, (0, 7)).reshape(1, 8)
    out8 = pl.pallas_call(
        _out_body,
        grid=(N // NB,),
        in_specs=[pl.BlockSpec((NB, D), lambda k: (k, 0)),
                  pl.BlockSpec((NB, 1), lambda k: (k, 0)),
                  _full((D, D)), _full((1, D)),
                  _full((D, 8)), _full((1, 8))],
        out_specs=pl.BlockSpec((G, 8), lambda k: (0, 0)),
        out_shape=jax.ShapeDtypeStruct((G, 8), f32),
    )(h, gids, Wu1, bu1.reshape(1, D), wu2p, bu2p)

    return out8[:, :1]


# speedup vs baseline: 1.4583x; 1.4583x over previous
import functools

import jax
import jax.numpy as jnp
from jax import lax
from jax.experimental import pallas as pl
from jax.experimental.pallas import tpu as pltpu
from jax.experimental.pallas import tpu_sc as plsc

N = 50000
E = 800000
G = 1024
D = 64
NCONV = 3
RBF = 5
GAP = 1.25
NB = 2000
EB2 = 2000
CH = 128
NCH = E // CH
NSUB = 16
RPT = 3128
NPAD = NSUB * RPT
CW = 16
CG = 4


def _sp(x, beta, thr):
    z = x * beta
    return jnp.where(z > thr, x,
                     (1.0 / beta) * jnp.log1p(jnp.exp(jnp.minimum(z, thr))))


def _emb_body(at_ref, emb_ref, w_ref, h_ref, nn_ref):
    at = at_ref[:]
    oh = (at == lax.broadcasted_iota(jnp.int32, (NB, 100), 1))
    h = jnp.dot(oh.astype(jnp.float32), emb_ref[:],
                preferred_element_type=jnp.float32)
    h_ref[:] = h
    nn = jnp.dot(h, w_ref[:], preferred_element_type=jnp.float32)
    for g in range(CG):
        nn_ref[g] = nn[:, g * CW:(g + 1) * CW]


def _eh_body(d_ref, w1b_ref, b1t_ref, w2b_ref, b2t_ref, out_ref):
    col = lax.broadcasted_iota(jnp.int32, (2, 16), 1)
    row = lax.broadcasted_iota(jnp.int32, (2, 16), 0)
    rep = ((col >> 3) == row).astype(jnp.float32)
    d16 = jnp.dot(d_ref[:], rep, preferred_element_type=jnp.float32)
    ctile = (lax.broadcasted_iota(jnp.int32, (1, 16), 1) & 7
             ).astype(jnp.float32) * GAP
    radial = d16 - ctile
    rbf = jnp.exp((-1.0 / GAP) * radial * radial)
    p = _sp(jnp.dot(rbf, w1b_ref[:], preferred_element_type=jnp.float32)
            + b1t_ref[:], 0.5, 14.0)
    eh = jnp.dot(p, w2b_ref[:], preferred_element_type=jnp.float32) + b2t_ref[:]
    out_ref[:] = eh


def _upd_body(a_ref, h_ref, wn2_ref, bn2_ref, wn3_ref, bn3_ref,
              hout_ref, wnext_ref=None, nnout_ref=None):
    a = jnp.concatenate([a_ref[g] for g in range(CG)], axis=1)
    cf1 = _sp(jnp.dot(a, wn2_ref[:], preferred_element_type=jnp.float32)
              + bn2_ref[:], 0.5, 14.0)
    hn = h_ref[:] + jnp.dot(cf1, wn3_ref[:],
                            preferred_element_type=jnp.float32) + bn3_ref[:]
    hout_ref[:] = hn
    if wnext_ref is not None:
        nn = jnp.dot(hn, wnext_ref[:], preferred_element_type=jnp.float32)
        for g in range(CG):
            nnout_ref[g] = nn[:, g * CW:(g + 1) * CW]


def _upd_next_body(a_ref, h_ref, wn2_ref, bn2_ref, wn3_ref, bn3_ref,
                   wnext_ref, hout_ref, nnout_ref):
    _upd_body(a_ref, h_ref, wn2_ref, bn2_ref, wn3_ref, bn3_ref,
              hout_ref, wnext_ref, nnout_ref)


def _upd_last_body(a_ref, h_ref, wn2_ref, bn2_ref, wn3_ref, bn3_ref, hout_ref):
    _upd_body(a_ref, h_ref, wn2_ref, bn2_ref, wn3_ref, bn3_ref, hout_ref)


def _out_body(h_ref, gid_ref, wu1_ref, bu1_ref, wu2_ref, bu2_ref, out_ref):
    i = pl.program_id(0)
    hh = _sp(jnp.dot(h_ref[:], wu1_ref[:], preferred_element_type=jnp.float32)
             + bu1_ref[:], 1.0, 20.0) - jnp.log(2.0)
    u = jnp.dot(hh, wu2_ref[:], preferred_element_type=jnp.float32) + bu2_ref[:]
    gid = gid_ref[:]
    oh = (gid == lax.broadcasted_iota(jnp.int32, (NB, G), 1))
    contrib = lax.dot_general(oh.astype(jnp.float32), u,
                              dimension_numbers=(((0,), (0,)), ((), ())),
                              preferred_element_type=jnp.float32)

    @pl.when(i == 0)
    def _():
        out_ref[:] = jnp.zeros_like(out_ref)

    out_ref[:] = out_ref[:] + contrib


NBUF = 8
ZR = 184


def _edge_body(nn, ehh, src2, dst2, out, sidx, didx, sdidx, rows, ehb, zbuf,
               agg, sem_ld, sem_g, sem_s):
    c = lax.axis_index("c")
    s = lax.axis_index("s")
    z16 = jnp.zeros((16,), jnp.float32)

    def zb(r, carry):
        zbuf[r, pl.ds(0, 16)] = z16
        return carry
    lax.fori_loop(0, ZR, zb, 0)

    base = NCH // NSUB
    extra = NCH - base * NSUB
    start = s * base + jnp.minimum(s, extra)
    cnt = base + (s < extra).astype(jnp.int32)
    n_sc = cnt // NBUF
    rem = cnt - n_sc * NBUF
    npairs = n_sc // 2
    n_pipe = npairs * 2

    def issue_loads(t, p):
        ch0 = start + t * NBUF
        pltpu.async_copy(src2.at[pl.ds(ch0, NBUF)], sidx.at[p], sem_ld)
        pltpu.async_copy(dst2.at[pl.ds(ch0, NBUF)], didx.at[p], sem_ld)
        pltpu.async_copy(ehh.at[pl.ds(ch0 * CH, NBUF * CH),
                                pl.ds(g * CW, CW)],
                         ehb.at[p], sem_ld)

    def wait_loads(p):
        pltpu.make_async_copy(src2.at[pl.ds(0, NBUF)], sidx.at[p],
                              sem_ld).wait()
        pltpu.make_async_copy(dst2.at[pl.ds(0, NBUF)], didx.at[p],
                              sem_ld).wait()
        pltpu.make_async_copy(ehh.at[pl.ds(0, NBUF * CH), pl.ds(0, CW)],
                              ehb.at[p], sem_ld).wait()

    def adjust(p):
        for b in range(NBUF):
            for j in range(CH // 16):
                sl = pl.ds(j * 16, 16)
                sidx[p, b, sl] = sidx[p, b, sl] + goff

    def issue_gathers(p):
        for b in range(NBUF):
            pltpu.async_copy(nn.at[sidx.at[p, b]], rows.at[p, b], sem_g)

    def wait_gathers(p):
        for b in range(NBUF):
            pltpu.make_async_copy(nn.at[sidx.at[p, b]], rows.at[p, b],
                                  sem_g).wait()

    def do_mul(p):
        sl = pl.ds(0, 16)

        def mul(r, cc):
            for b in range(NBUF):
                rows[p, b, r, sl] = (rows[p, b, r, sl]
                                     * ehb[p, b * CH + r, sl])
            return cc
        lax.fori_loop(0, CH, mul, 0)

    def issue_scatters(p):
        for b in range(NBUF):
            for j in range(CH // 16):
                sl = pl.ds(j * 16, 16)
                sdidx[p, b, sl] = didx[p, b, sl]
        for b in range(NBUF):
            pltpu.async_copy(rows.at[p, b], agg.at[sdidx.at[p, b]], sem_s,
                             add=True)

    def drain_scatters(p):
        for b in range(NBUF):
            pltpu.make_async_copy(rows.at[p, b], agg.at[sdidx.at[p, b]],
                                  sem_s).wait()

    def step(t, p):
        wait_gathers(p)
        do_mul(p)
        issue_scatters(p)

        @pl.when(t + 1 < n_pipe)
        def _():
            wait_loads(1 - p)
            adjust(1 - p)

        @pl.when(t >= 1)
        def _():
            drain_scatters(1 - p)

        @pl.when(t + 1 < n_pipe)
        def _():
            issue_gathers(1 - p)

        @pl.when(t + 2 < n_pipe)
        def _():
            issue_loads(t + 2, p)

    def do_chunks(ch0, nb):
        eo0 = ch0 * CH
        pltpu.sync_copy(src2.at[pl.ds(ch0, nb)], sidx.at[0, pl.ds(0, nb)])
        pltpu.sync_copy(dst2.at[pl.ds(ch0, nb)], didx.at[0, pl.ds(0, nb)])
        pltpu.sync_copy(ehh.at[pl.ds(eo0, nb * CH), pl.ds(g * CW, CW)],
                        ehb.at[0, pl.ds(0, nb * CH)])
        for b in range(nb):
            for j in range(CH // 16):
                sl = pl.ds(j * 16, 16)
                sidx[0, b, sl] = sidx[0, b, sl] + goff
        gds = [pltpu.async_copy(nn.at[sidx.at[0, b]], rows.at[0, b], sem_g)
               for b in range(nb)]
        for d in gds:
            d.wait()
        sl16 = pl.ds(0, 16)
        for b in range(nb):
            def mul(r, cc):
                rows[0, b, r, sl16] = (rows[0, b, r, sl16]
                                       * ehb[0, b * CH + r, sl16])
                return cc
            lax.fori_loop(0, CH, mul, 0)
        sds = [pltpu.async_copy(rows.at[0, b], agg.at[didx.at[0, b]], sem_s,
                                add=True)
               for b in range(nb)]
        for d in sds:
            d.wait()

    for cp in range(CG // 2):
        g = c * (CG // 2) + cp
        goff = g * N
        ooff = g * NPAD

        def zs(k, carry):
            pltpu.sync_copy(zbuf, agg.at[pl.ds(s * RPT + k * ZR, ZR)])
            return carry
        lax.fori_loop(0, RPT // ZR, zs, 0)
        plsc.subcore_barrier()

        issue_loads(0, 0)
        wait_loads(0)
        adjust(0)
        issue_gathers(0)

        @pl.when(n_pipe > 1)
        def _():
            issue_loads(1, 1)

        def per_pair(q, carry):
            step(2 * q, 0)
            step(2 * q + 1, 1)
            return carry
        lax.fori_loop(0, npairs, per_pair, 0)
        drain_scatters(1)

        def per_odd(t, carry):
            do_chunks(start + n_pipe * NBUF, NBUF)
            return carry
        lax.fori_loop(0, n_sc - n_pipe, per_odd, 0)

        def per_tail(t, carry):
            do_chunks(start + n_sc * NBUF + t, 1)
            return carry
        lax.fori_loop(0, rem, per_tail, 0)

        plsc.subcore_barrier()
        pltpu.sync_copy(agg.at[pl.ds(s * RPT, RPT)],
                        out.at[pl.ds(ooff + s * RPT, RPT)])


_edge_pass = functools.partial(
    pl.kernel,
    out_type=jax.ShapeDtypeStruct((CG * NPAD, CW), jnp.float32),
    mesh=plsc.VectorSubcoreMesh(core_axis_name="c", subcore_axis_name="s",
                                num_cores=2, num_subcores=NSUB),
    compiler_params=pltpu.CompilerParams(use_tc_tiling_on_sc=False),
    scratch_types=[
        pltpu.VMEM((2, NBUF, CH), jnp.int32),
        pltpu.VMEM((2, NBUF, CH), jnp.int32),
        pltpu.VMEM((2, NBUF, CH), jnp.int32),
        pltpu.VMEM((2, NBUF, CH, CW), jnp.float32),
        pltpu.VMEM((2, NBUF * CH, CW), jnp.float32),
        pltpu.VMEM((ZR, CW), jnp.float32),
        pltpu.VMEM_SHARED((NPAD, CW), jnp.float32),
        pltpu.SemaphoreType.DMA,
        pltpu.SemaphoreType.DMA,
        pltpu.SemaphoreType.DMA,
    ],
)(_edge_body)


def _full(shape):
    return pl.BlockSpec(shape, lambda i: tuple(0 for _ in shape))


def kernel(atom_types, edge_distances, edge_index, node_graph_ids, emb,
           Wn1, Wc1, bc1, Wc2, bc2, Wn2, bn2, Wn3, bn3, Wu1, bu1, Wu2, bu2):
    f32 = jnp.float32
    src = edge_index[0].astype(jnp.int32)
    dst = edge_index[1].astype(jnp.int32)
    at = atom_types.astype(jnp.int32).reshape(N, 1)
    gids = node_graph_ids.astype(jnp.int32).reshape(N, 1)

    h, nn = pl.pallas_call(
        _emb_body,
        grid=(N // NB,),
        in_specs=[pl.BlockSpec((NB, 1), lambda i: (i, 0)),
                  _full((100, D)),
                  _full((D, D))],
        out_specs=[pl.BlockSpec((NB, D), lambda i: (i, 0)),
                   pl.BlockSpec((CG, NB, CW), lambda i: (0, i, 0))],
        out_shape=[jax.ShapeDtypeStruct((N, D), f32),
                   jax.ShapeDtypeStruct((CG, N, CW), f32)],
    )(at, emb, Wn1[0])

    d2 = edge_distances.reshape(E // 2, 2)
    z16 = jnp.zeros((16, 2 * D), f32)
    z128 = jnp.zeros((2 * D, 2 * D), f32)
    for i in range(NCONV):
        w1b = z16.at[0:RBF, 0:D].set(Wc1[i]).at[8:8 + RBF, D:2 * D].set(Wc1[i])
        w2b = z128.at[0:D, 0:D].set(Wc2[i]).at[D:2 * D, D:2 * D].set(Wc2[i])
        b1t = jnp.concatenate([bc1[i], bc1[i]]).reshape(1, 2 * D)
        b2t = jnp.concatenate([bc2[i], bc2[i]]).reshape(1, 2 * D)
        eh2 = pl.pallas_call(
            _eh_body,
            grid=(E // 2 // EB2,),
            in_specs=[pl.BlockSpec((EB2, 2), lambda k: (k, 0)),
                      _full((16, 2 * D)),
                      _full((1, 2 * D)),
                      _full((2 * D, 2 * D)),
                      _full((1, 2 * D))],
            out_specs=pl.BlockSpec((EB2, 2 * D), lambda k: (k, 0)),
            out_shape=jax.ShapeDtypeStruct((E // 2, 2 * D), f32),
        )(d2, w1b, b1t, w2b, b2t)
        eh = eh2.reshape(E, D)

        agg_flat = _edge_pass(nn.reshape(CG * N, CW), eh,
                              src.reshape(NCH, CH), dst.reshape(NCH, CH))
        agg = agg_flat.reshape(CG, NPAD, CW)

        if i + 1 < NCONV:
            h, nn = pl.pallas_call(
                _upd_next_body,
                grid=(N // NB,),
                in_specs=[pl.BlockSpec((CG, NB, CW), lambda k: (0, k, 0)),
                          pl.BlockSpec((NB, D), lambda k: (k, 0)),
                          _full((D, D)), _full((1, D)),
                          _full((D, D)), _full((1, D)),
                          _full((D, D))],
                out_specs=[pl.BlockSpec((NB, D), lambda k: (k, 0)),
                           pl.BlockSpec((CG, NB, CW), lambda k: (0, k, 0))],
                out_shape=[jax.ShapeDtypeStruct((N, D), f32),
                           jax.ShapeDtypeStruct((CG, N, CW), f32)],
            )(agg, h, Wn2[i], bn2[i].reshape(1, D),
              Wn3[i], bn3[i].reshape(1, D), Wn1[i + 1])
        else:
            h = pl.pallas_call(
                _upd_last_body,
                grid=(N // NB,),
                in_specs=[pl.BlockSpec((CG, NB, CW), lambda k: (0, k, 0)),
                          pl.BlockSpec((NB, D), lambda k: (k, 0)),
                          _full((D, D)), _full((1, D)),
                          _full((D, D)), _full((1, D))],
                out_specs=pl.BlockSpec((NB, D), lambda k: (k, 0)),
                out_shape=jax.ShapeDtypeStruct((N, D), f32),
            )(agg, h, Wn2[i], bn2[i].reshape(1, D),
              Wn3[i], bn3[i].reshape(1, D))

    wu2p = jnp.pad(Wu2, ((0, 0), (0, 7)))
    bu2p = jnp.pad(bu2, (0, 7)).reshape(1, 8)
    out8 = pl.pallas_call(
        _out_body,
        grid=(N // NB,),
        in_specs=[pl.BlockSpec((NB, D), lambda k: (k, 0)),
                  pl.BlockSpec((NB, 1), lambda k: (k, 0)),
                  _full((D, D)), _full((1, D)),
                  _full((D, 8)), _full((1, 8))],
        out_specs=pl.BlockSpec((G, 8), lambda k: (0, 0)),
        out_shape=jax.ShapeDtypeStruct((G, 8), f32),
    )(h, gids, Wu1, bu1.reshape(1, D), wu2p, bu2p)

    return out8[:, :1]
